# Initial kernel scaffold; baseline (speedup 1.0000x reference)
import functools

import jax
import jax.numpy as jnp
from jax import lax
from jax.experimental import pallas as pl
from jax.experimental.pallas import tpu as pltpu
from jax.experimental.pallas import tpu_sc as plsc

N = 10000
E = 320000
D = 128

NC = 2
NS = 16
NW = NC * NS
LANES = 16

CHUNK = 128
CPW = -(-E // (NW * CHUNK))
PW = CPW * CHUNK
EP = PW * NW

RPS = 626
NP = RPS * NS

_mesh = plsc.VectorSubcoreMesh(core_axis_name="c", subcore_axis_name="s")


def _sc_agg_body(with_deg, x_hbm, src_hbm, dst_hbm, *refs):
    if with_deg:
        (agg_out, deg_out, sidx, didx, rows, ones_v, zbuf, zdeg,
         agg_sh, deg_sh) = refs
    else:
        agg_out, sidx, didx, rows, zbuf, agg_sh = refs

    cid = lax.axis_index("c")
    sid = lax.axis_index("s")
    wid = cid * NS + sid

    zero16 = jnp.zeros((LANES,), jnp.float32)

    @pl.loop(0, zbuf.shape[0])
    def _(r):
        @pl.loop(0, D // LANES)
        def _(g):
            zbuf[r, pl.ds(g * LANES, LANES)] = zero16

    if with_deg:
        @pl.loop(0, zdeg.shape[0])
        def _(r):
            zdeg[r, pl.ds(0, LANES)] = zero16

        ones16 = jnp.ones((LANES,), jnp.float32)

        @pl.loop(0, CHUNK)
        def _(r):
            ones_v[r, pl.ds(0, LANES)] = ones16

    r0 = sid * RPS
    nfull = RPS // zbuf.shape[0]
    rem = RPS - nfull * zbuf.shape[0]
    for j in range(nfull):
        pltpu.sync_copy(zbuf, agg_sh.at[pl.ds(r0 + j * zbuf.shape[0],
                                              zbuf.shape[0])])
    if rem:
        pltpu.sync_copy(zbuf.at[pl.ds(0, rem)],
                        agg_sh.at[pl.ds(r0 + nfull * zbuf.shape[0], rem)])
    if with_deg:
        pltpu.sync_copy(zdeg.at[pl.ds(0, RPS)], deg_sh.at[pl.ds(r0, RPS)])

    plsc.subcore_barrier()

    base0 = wid * PW

    @pl.loop(0, CPW)
    def _(i):
        base = base0 + i * CHUNK
        pltpu.sync_copy(src_hbm.at[pl.ds(base, CHUNK)], sidx)
        pltpu.sync_copy(dst_hbm.at[pl.ds(base, CHUNK)], didx)
        pltpu.sync_copy(x_hbm.at[sidx], rows)
        pltpu.sync_copy(rows, agg_sh.at[didx], add=True)
        if with_deg:
            pltpu.sync_copy(ones_v, deg_sh.at[didx], add=True)

    plsc.subcore_barrier()

    pltpu.sync_copy(agg_sh.at[pl.ds(r0, RPS)],
                    agg_out.at[cid, pl.ds(r0, RPS)])
    if with_deg:
        pltpu.sync_copy(deg_sh.at[pl.ds(r0, RPS)],
                        deg_out.at[cid, pl.ds(r0, RPS)])


_AGG_OUT = jax.ShapeDtypeStruct((NC, NP, D), jnp.float32)
_DEG_OUT = jax.ShapeDtypeStruct((NC, NP, LANES), jnp.float32)

_sc_agg_deg = pl.kernel(
    functools.partial(_sc_agg_body, True),
    out_type=[_AGG_OUT, _DEG_OUT],
    mesh=_mesh,
    scratch_types=[
        pltpu.VMEM((CHUNK,), jnp.int32),
        pltpu.VMEM((CHUNK,), jnp.int32),
        pltpu.VMEM((CHUNK, D), jnp.float32),
        pltpu.VMEM((CHUNK, LANES), jnp.float32),
        pltpu.VMEM((128, D), jnp.float32),
        pltpu.VMEM((RPS, LANES), jnp.float32),
        pltpu.VMEM_SHARED((NP, D), jnp.float32),
        pltpu.VMEM_SHARED((NP, LANES), jnp.float32),
    ],
    name="sc_agg_deg",
)

_sc_agg = pl.kernel(
    functools.partial(_sc_agg_body, False),
    out_type=_AGG_OUT,
    mesh=_mesh,
    scratch_types=[
        pltpu.VMEM((CHUNK,), jnp.int32),
        pltpu.VMEM((CHUNK,), jnp.int32),
        pltpu.VMEM((CHUNK, D), jnp.float32),
        pltpu.VMEM((128, D), jnp.float32),
        pltpu.VMEM_SHARED((NP, D), jnp.float32),
    ],
    name="sc_agg",
)


_BLK = 2000


def _combine_body(relu, x_ref, agg_ref, deg_ref, ws_ref, wn_ref, b_ref,
                  o_ref):
    agg = agg_ref[0] + agg_ref[1]
    deg = deg_ref[0, :, 0:1] + deg_ref[1, :, 0:1]
    hn = agg / jnp.maximum(deg, 1.0)
    h = (jnp.dot(x_ref[...], ws_ref[...], preferred_element_type=jnp.float32)
         + jnp.dot(hn, wn_ref[...], preferred_element_type=jnp.float32)
         + b_ref[...])
    o_ref[...] = jnp.maximum(h, 0.0) if relu else h


def _combine(x, agg, deg, w_self, w_neigh, b, relu):
    return pl.pallas_call(
        functools.partial(_combine_body, relu),
        grid=(N // _BLK,),
        in_specs=[
            pl.BlockSpec((_BLK, D), lambda i: (i, 0)),
            pl.BlockSpec((NC, _BLK, D), lambda i: (0, i, 0)),
            pl.BlockSpec((NC, _BLK, LANES), lambda i: (0, i, 0)),
            pl.BlockSpec((D, D), lambda i: (0, 0)),
            pl.BlockSpec((D, D), lambda i: (0, 0)),
            pl.BlockSpec((1, D), lambda i: (0, 0)),
        ],
        out_specs=pl.BlockSpec((_BLK, D), lambda i: (i, 0)),
        out_shape=jax.ShapeDtypeStruct((N, D), jnp.float32),
    )(x, agg, deg, w_self, w_neigh, b.reshape(1, D))


def kernel(in_feat, edge_index, W_self1, W_neigh1, b1, W_self2, W_neigh2,
           b2):
    src = edge_index[0].astype(jnp.int32)
    dst = edge_index[1].astype(jnp.int32)
    pad = EP - E
    src_p = jnp.concatenate([src, jnp.zeros((pad,), jnp.int32)])
    dst_p = jnp.concatenate([dst, jnp.full((pad,), N, jnp.int32)])

    agg1, deg = _sc_agg_deg(in_feat, src_p, dst_p)
    h1 = _combine(in_feat, agg1, deg, W_self1, W_neigh1, b1, relu=True)
    agg2 = _sc_agg(h1, src_p, dst_p)
    return _combine(h1, agg2, deg, W_self2, W_neigh2, b2, relu=False)


# speedup vs baseline: 3.5130x; 3.5130x over previous
import functools

import jax
import jax.numpy as jnp
from jax import lax
from jax.experimental import pallas as pl
from jax.experimental.pallas import tpu as pltpu
from jax.experimental.pallas import tpu_sc as plsc

N = 10000
E = 320000
D = 128
DH = D // 2

NC = 2
NS = 16
NW = NC * NS
LANES = 16

CHUNK = 128
CPW = -(-E // (NW * CHUNK))
PW = CPW * CHUNK
EP = PW * NW

RPS = 632
NP = RPS * NS

_mesh = plsc.VectorSubcoreMesh(core_axis_name="c", subcore_axis_name="s")
_sc_params = pltpu.CompilerParams(use_tc_tiling_on_sc=False)


def _sc_agg_body(with_deg, x_hbm, src0_hbm, src1_hbm, dst_hbm, *refs):
    if with_deg:
        (agg_out, deg_out, sidx, didx, rows, ones_v, zbuf, zdeg,
         agg_sh, deg_sh) = refs
    else:
        agg_out, sidx, didx, rows, zbuf, agg_sh = refs

    cid = lax.axis_index("c")
    sid = lax.axis_index("s")
    wid = cid * NS + sid

    zero16 = jnp.zeros((LANES,), jnp.float32)

    @pl.loop(0, zbuf.shape[0])
    def _(r):
        @pl.loop(0, DH // LANES)
        def _(g):
            zbuf[r, pl.ds(g * LANES, LANES)] = zero16

    if with_deg:
        @pl.loop(0, zdeg.shape[0])
        def _(r):
            zdeg[r, pl.ds(0, LANES)] = zero16

        ones16 = jnp.ones((LANES,), jnp.float32)

        @pl.loop(0, CHUNK)
        def _(r):
            ones_v[r, pl.ds(0, LANES)] = ones16

    r0 = sid * RPS
    zrows = zbuf.shape[0]
    nfull, rem = RPS // zrows, RPS % zrows
    base0 = wid * PW

    for h, src_hbm in ((0, src0_hbm), (1, src1_hbm)):
        for j in range(nfull):
            pltpu.sync_copy(zbuf, agg_sh.at[pl.ds(r0 + j * zrows, zrows)])
        if rem:
            pltpu.sync_copy(zbuf.at[pl.ds(0, rem)],
                            agg_sh.at[pl.ds(r0 + nfull * zrows, rem)])
        if with_deg and h == 0:
            pltpu.sync_copy(zdeg.at[pl.ds(0, RPS)], deg_sh.at[pl.ds(r0, RPS)])

        plsc.subcore_barrier()

        @pl.loop(0, CPW)
        def _(i):
            base = base0 + i * CHUNK
            pltpu.sync_copy(src_hbm.at[pl.ds(base, CHUNK)], sidx)
            pltpu.sync_copy(dst_hbm.at[pl.ds(base, CHUNK)], didx)
            pltpu.sync_copy(x_hbm.at[sidx], rows)
            pltpu.sync_copy(rows, agg_sh.at[didx], add=True)
            if with_deg and h == 0:
                pltpu.sync_copy(ones_v, deg_sh.at[didx], add=True)

        plsc.subcore_barrier()

        pltpu.sync_copy(agg_sh.at[pl.ds(r0, RPS)],
                        agg_out.at[h, cid, pl.ds(r0, RPS)])
        if with_deg and h == 0:
            pltpu.sync_copy(deg_sh.at[pl.ds(r0, RPS)],
                            deg_out.at[cid, pl.ds(r0, RPS)])


_AGG_OUT = jax.ShapeDtypeStruct((2, NC, NP, DH), jnp.float32)
_DEG_OUT = jax.ShapeDtypeStruct((NC, NP, LANES), jnp.float32)

_sc_agg_deg = pl.kernel(
    functools.partial(_sc_agg_body, True),
    out_type=[_AGG_OUT, _DEG_OUT],
    mesh=_mesh,
    scratch_types=[
        pltpu.VMEM((CHUNK,), jnp.int32),
        pltpu.VMEM((CHUNK,), jnp.int32),
        pltpu.VMEM((CHUNK, DH), jnp.float32),
        pltpu.VMEM((CHUNK, LANES), jnp.float32),
        pltpu.VMEM((128, DH), jnp.float32),
        pltpu.VMEM((RPS, LANES), jnp.float32),
        pltpu.VMEM_SHARED((NP, DH), jnp.float32),
        pltpu.VMEM_SHARED((NP, LANES), jnp.float32),
    ],
    compiler_params=_sc_params,
    name="sc_agg_deg",
)

_sc_agg = pl.kernel(
    functools.partial(_sc_agg_body, False),
    out_type=_AGG_OUT,
    mesh=_mesh,
    scratch_types=[
        pltpu.VMEM((CHUNK,), jnp.int32),
        pltpu.VMEM((CHUNK,), jnp.int32),
        pltpu.VMEM((CHUNK, DH), jnp.float32),
        pltpu.VMEM((128, DH), jnp.float32),
        pltpu.VMEM_SHARED((NP, DH), jnp.float32),
    ],
    compiler_params=_sc_params,
    name="sc_agg",
)


_BLK = 2000


def _combine_body(relu, x_ref, agg_ref, deg_ref, ws_ref, wn_ref, b_ref,
                  o_ref):
    lo = agg_ref[0, 0] + agg_ref[0, 1]
    hi = agg_ref[1, 0] + agg_ref[1, 1]
    agg = jnp.concatenate([lo, hi], axis=1)
    deg = deg_ref[0, :, 0:1] + deg_ref[1, :, 0:1]
    hn = agg / jnp.maximum(deg, 1.0)
    h = (jnp.dot(x_ref[...], ws_ref[...], preferred_element_type=jnp.float32)
         + jnp.dot(hn, wn_ref[...], preferred_element_type=jnp.float32)
         + b_ref[...])
    o_ref[...] = jnp.maximum(h, 0.0) if relu else h


def _combine(x, agg, deg, w_self, w_neigh, b, relu):
    return pl.pallas_call(
        functools.partial(_combine_body, relu),
        grid=(N // _BLK,),
        in_specs=[
            pl.BlockSpec((_BLK, D), lambda i: (i, 0)),
            pl.BlockSpec((2, NC, _BLK, DH), lambda i: (0, 0, i, 0)),
            pl.BlockSpec((NC, _BLK, LANES), lambda i: (0, i, 0)),
            pl.BlockSpec((D, D), lambda i: (0, 0)),
            pl.BlockSpec((D, D), lambda i: (0, 0)),
            pl.BlockSpec((1, D), lambda i: (0, 0)),
        ],
        out_specs=pl.BlockSpec((_BLK, D), lambda i: (i, 0)),
        out_shape=jax.ShapeDtypeStruct((N, D), jnp.float32),
    )(x, agg, deg, w_self, w_neigh, b.reshape(1, D))


def kernel(in_feat, edge_index, W_self1, W_neigh1, b1, W_self2, W_neigh2,
           b2):
    src = edge_index[0].astype(jnp.int32)
    dst = edge_index[1].astype(jnp.int32)
    pad = EP - E
    src_p = jnp.concatenate([src, jnp.zeros((pad,), jnp.int32)])
    dst_p = jnp.concatenate([dst, jnp.full((pad,), N, jnp.int32)])
    src0 = src_p * 2
    src1 = src0 + 1

    x2 = in_feat.reshape(2 * N, DH)
    agg1, deg = _sc_agg_deg(x2, src0, src1, dst_p)
    h1 = _combine(in_feat, agg1, deg, W_self1, W_neigh1, b1, relu=True)
    agg2 = _sc_agg(h1.reshape(2 * N, DH), src0, src1, dst_p)
    return _combine(h1, agg2, deg, W_self2, W_neigh2, b2, relu=False)


# speedup vs baseline: 3.8795x; 1.1043x over previous
import functools

import jax
import jax.numpy as jnp
from jax import lax
from jax.experimental import pallas as pl
from jax.experimental.pallas import tpu as pltpu
from jax.experimental.pallas import tpu_sc as plsc

N = 10000
E = 320000
D = 128
DH = D // 2

NC = 2
NS = 16
NW = NC * NS
LANES = 16

CHUNK = 128
CPW = 80
PW = CPW * CHUNK
EP = PW * NW

RPS = 632
NP = RPS * NS

_mesh = plsc.VectorSubcoreMesh(core_axis_name="c", subcore_axis_name="s")
_sc_params = pltpu.CompilerParams(use_tc_tiling_on_sc=False)


def _sc_agg_body(with_deg, x_hbm, src0_hbm, src1_hbm, dst_hbm, *refs):
    if with_deg:
        (agg_out, deg_out, sidx_all, didx_all, rows0, rows1, ones_v, zbuf,
         zdeg, agg_sh, deg_sh, sem0, sem1) = refs
    else:
        (agg_out, sidx_all, didx_all, rows0, rows1, zbuf, agg_sh,
         sem0, sem1) = refs

    cid = lax.axis_index("c")
    sid = lax.axis_index("s")
    wid = cid * NS + sid

    zero16 = jnp.zeros((LANES,), jnp.float32)

    @pl.loop(0, zbuf.shape[0])
    def _(r):
        @pl.loop(0, DH // LANES)
        def _(g):
            zbuf[r, pl.ds(g * LANES, LANES)] = zero16

    if with_deg:
        @pl.loop(0, zdeg.shape[0])
        def _(r):
            zdeg[r, pl.ds(0, LANES)] = zero16

        ones16 = jnp.ones((LANES,), jnp.float32)

        @pl.loop(0, CHUNK)
        def _(r):
            ones_v[r, pl.ds(0, LANES)] = ones16

    r0 = sid * RPS
    zrows = zbuf.shape[0]
    nfull, rem = RPS // zrows, RPS % zrows

    pltpu.sync_copy(dst_hbm.at[wid], didx_all)

    def _gather(i, rows, sem):
        return pltpu.make_async_copy(x_hbm.at[sidx_all.at[i]], rows, sem)

    for h, src_hbm in ((0, src0_hbm), (1, src1_hbm)):
        for j in range(nfull):
            pltpu.sync_copy(zbuf, agg_sh.at[pl.ds(r0 + j * zrows, zrows)])
        if rem:
            pltpu.sync_copy(zbuf.at[pl.ds(0, rem)],
                            agg_sh.at[pl.ds(r0 + nfull * zrows, rem)])
        if with_deg and h == 0:
            pltpu.sync_copy(zdeg.at[pl.ds(0, RPS)], deg_sh.at[pl.ds(r0, RPS)])

        pltpu.sync_copy(src_hbm.at[wid], sidx_all)
        plsc.subcore_barrier()

        _gather(0, rows0, sem0).start()
        _gather(1, rows1, sem1).start()

        @pl.loop(0, CPW // 2)
        def _(j):
            i0 = 2 * j
            for i, rows, sem in ((i0, rows0, sem0), (i0 + 1, rows1, sem1)):
                _gather(i, rows, sem).wait()
                pltpu.sync_copy(rows, agg_sh.at[didx_all.at[i]], add=True)
                if with_deg and h == 0:
                    pltpu.sync_copy(ones_v, deg_sh.at[didx_all.at[i]],
                                    add=True)
                _gather(jnp.minimum(i + 2, CPW - 1), rows, sem).start()

        _gather(0, rows0, sem0).wait()
        _gather(0, rows1, sem1).wait()

        plsc.subcore_barrier()

        pltpu.sync_copy(agg_sh.at[pl.ds(r0, RPS)],
                        agg_out.at[h, cid, pl.ds(r0, RPS)])
        if with_deg and h == 0:
            pltpu.sync_copy(deg_sh.at[pl.ds(r0, RPS)],
                            deg_out.at[cid, pl.ds(r0, RPS)])


_AGG_OUT = jax.ShapeDtypeStruct((2, NC, NP, DH), jnp.float32)
_DEG_OUT = jax.ShapeDtypeStruct((NC, NP, LANES), jnp.float32)

_sc_agg_deg = pl.kernel(
    functools.partial(_sc_agg_body, True),
    out_type=[_AGG_OUT, _DEG_OUT],
    mesh=_mesh,
    scratch_types=[
        pltpu.VMEM((CPW, CHUNK), jnp.int32),
        pltpu.VMEM((CPW, CHUNK), jnp.int32),
        pltpu.VMEM((CHUNK, DH), jnp.float32),
        pltpu.VMEM((CHUNK, DH), jnp.float32),
        pltpu.VMEM((CHUNK, LANES), jnp.float32),
        pltpu.VMEM((128, DH), jnp.float32),
        pltpu.VMEM((RPS, LANES), jnp.float32),
        pltpu.VMEM_SHARED((NP, DH), jnp.float32),
        pltpu.VMEM_SHARED((NP, LANES), jnp.float32),
        pltpu.SemaphoreType.DMA,
        pltpu.SemaphoreType.DMA,
    ],
    compiler_params=_sc_params,
    name="sc_agg_deg",
)

_sc_agg = pl.kernel(
    functools.partial(_sc_agg_body, False),
    out_type=_AGG_OUT,
    mesh=_mesh,
    scratch_types=[
        pltpu.VMEM((CPW, CHUNK), jnp.int32),
        pltpu.VMEM((CPW, CHUNK), jnp.int32),
        pltpu.VMEM((CHUNK, DH), jnp.float32),
        pltpu.VMEM((CHUNK, DH), jnp.float32),
        pltpu.VMEM((128, DH), jnp.float32),
        pltpu.VMEM_SHARED((NP, DH), jnp.float32),
        pltpu.SemaphoreType.DMA,
        pltpu.SemaphoreType.DMA,
    ],
    compiler_params=_sc_params,
    name="sc_agg",
)


_BLK = 2000


def _combine_body(relu, x_ref, agg_ref, deg_ref, ws_ref, wn_ref, b_ref,
                  o_ref):
    lo = agg_ref[0, 0] + agg_ref[0, 1]
    hi = agg_ref[1, 0] + agg_ref[1, 1]
    agg = jnp.concatenate([lo, hi], axis=1)
    deg = deg_ref[0, :, 0:1] + deg_ref[1, :, 0:1]
    hn = agg / jnp.maximum(deg, 1.0)
    h = (jnp.dot(x_ref[...], ws_ref[...], preferred_element_type=jnp.float32)
         + jnp.dot(hn, wn_ref[...], preferred_element_type=jnp.float32)
         + b_ref[...])
    o_ref[...] = jnp.maximum(h, 0.0) if relu else h


def _combine(x, agg, deg, w_self, w_neigh, b, relu):
    return pl.pallas_call(
        functools.partial(_combine_body, relu),
        grid=(N // _BLK,),
        in_specs=[
            pl.BlockSpec((_BLK, D), lambda i: (i, 0)),
            pl.BlockSpec((2, NC, _BLK, DH), lambda i: (0, 0, i, 0)),
            pl.BlockSpec((NC, _BLK, LANES), lambda i: (0, i, 0)),
            pl.BlockSpec((D, D), lambda i: (0, 0)),
            pl.BlockSpec((D, D), lambda i: (0, 0)),
            pl.BlockSpec((1, D), lambda i: (0, 0)),
        ],
        out_specs=pl.BlockSpec((_BLK, D), lambda i: (i, 0)),
        out_shape=jax.ShapeDtypeStruct((N, D), jnp.float32),
    )(x, agg, deg, w_self, w_neigh, b.reshape(1, D))


def kernel(in_feat, edge_index, W_self1, W_neigh1, b1, W_self2, W_neigh2,
           b2):
    src = edge_index[0].astype(jnp.int32)
    dst = edge_index[1].astype(jnp.int32)
    pad = EP - E
    src_p = jnp.concatenate([src, jnp.zeros((pad,), jnp.int32)])
    dst_p = jnp.concatenate([dst, jnp.full((pad,), N, jnp.int32)])
    src0 = (src_p * 2).reshape(NW, CPW, CHUNK)
    src1 = (src_p * 2 + 1).reshape(NW, CPW, CHUNK)
    dst_p = dst_p.reshape(NW, CPW, CHUNK)

    x2 = in_feat.reshape(2 * N, DH)
    agg1, deg = _sc_agg_deg(x2, src0, src1, dst_p)
    h1 = _combine(in_feat, agg1, deg, W_self1, W_neigh1, b1, relu=True)
    agg2 = _sc_agg(h1.reshape(2 * N, DH), src0, src1, dst_p)
    return _combine(h1, agg2, deg, W_self2, W_neigh2, b2, relu=False)
